# Initial kernel scaffold; baseline (speedup 1.0000x reference)
import functools

import jax
import jax.numpy as jnp
from jax import lax
from jax.experimental import pallas as pl
from jax.experimental.pallas import tpu as pltpu
from jax.experimental.pallas import tpu_sc as plsc

B = 16384
S = 100
D1 = 512
D2 = 256

NC = 2
NS = 16
NW = NC * NS
ROWS_W = B // NW
CH = 128
NCHUNK = ROWS_W // CH

_sc_mesh = plsc.VectorSubcoreMesh(core_axis_name="c", subcore_axis_name="s")


@functools.partial(
    pl.kernel,
    mesh=_sc_mesh,
    out_type=(
        jax.ShapeDtypeStruct((B, S), jnp.float32),
        jax.ShapeDtypeStruct((B, S), jnp.float32),
    ),
    scratch_types=[
        pltpu.VMEM((CH, S), jnp.int32),
        pltpu.VMEM((CH, S), jnp.float32),
        pltpu.VMEM((CH, S), jnp.float32),
        pltpu.SemaphoreType.DMA,
        pltpu.SemaphoreType.DMA,
    ],
)
def _sc_gather(idx_hbm, sparse_hbm, certain_hbm, out_s, out_c,
               idx_v, sbuf, cbuf, sem_s, sem_c):
    wid = lax.axis_index("s") * NC + lax.axis_index("c")
    for k in range(NCHUNK):
        r0 = wid * ROWS_W + k * CH
        pltpu.sync_copy(idx_hbm.at[pl.ds(r0, CH)], idx_v)
        a = pltpu.async_copy(sparse_hbm.at[idx_v], sbuf, sem_s)
        b = pltpu.async_copy(certain_hbm.at[idx_v], cbuf, sem_c)
        a.wait()
        b.wait()
        pltpu.sync_copy(sbuf, out_s.at[pl.ds(r0, CH)])
        pltpu.sync_copy(cbuf, out_c.at[pl.ds(r0, CH)])


BM = 1024


def _tower_body(x_ref, c_ref, lab_ref, gb_ref, w1_ref, b1_ref, w2_ref,
                b2_ref, w3_ref, pred_ref, loss_ref, acc_ref):
    i = pl.program_id(0)
    x = x_ref[...]
    h = jnp.dot(x, w1_ref[...], preferred_element_type=jnp.float32)
    h = jnp.maximum(h + b1_ref[...], 0.0)
    h = jnp.dot(h, w2_ref[...], preferred_element_type=jnp.float32)
    h = jnp.maximum(h + b2_ref[...], 0.0)
    nn = jnp.sum(h * w3_ref[...], axis=1)
    logits = jnp.sum(x, axis=1) + nn + gb_ref[0, 0]
    pred_ref[...] = jax.nn.sigmoid(logits)
    craw = jax.nn.sigmoid(jnp.sum(c_ref[...], axis=1)) + 0.5
    xent = (jnp.maximum(logits, 0.0) - logits * lab_ref[...]
            + jnp.log1p(jnp.exp(-jnp.abs(logits))))
    s0 = jnp.sum(xent * craw)
    s1 = jnp.sum(craw)

    @pl.when(i == 0)
    def _():
        acc_ref[0] = 0.0
        acc_ref[1] = 0.0

    acc_ref[0] += s0
    acc_ref[1] += s1

    @pl.when(i == pl.num_programs(0) - 1)
    def _():
        loss_ref[0, 0] = B * acc_ref[0] / acc_ref[1]


_tower = pl.pallas_call(
    _tower_body,
    grid=(B // BM,),
    in_specs=[
        pl.BlockSpec((BM, S), lambda i: (i, 0)),
        pl.BlockSpec((BM, S), lambda i: (i, 0)),
        pl.BlockSpec((BM,), lambda i: (i,)),
        pl.BlockSpec(memory_space=pltpu.SMEM),
        pl.BlockSpec((S, D1), lambda i: (0, 0)),
        pl.BlockSpec((1, D1), lambda i: (0, 0)),
        pl.BlockSpec((D1, D2), lambda i: (0, 0)),
        pl.BlockSpec((1, D2), lambda i: (0, 0)),
        pl.BlockSpec((1, D2), lambda i: (0, 0)),
    ],
    out_specs=[
        pl.BlockSpec((BM,), lambda i: (i,)),
        pl.BlockSpec((1, 1), lambda i: (0, 0)),
    ],
    out_shape=[
        jax.ShapeDtypeStruct((B,), jnp.float32),
        jax.ShapeDtypeStruct((1, 1), jnp.float32),
    ],
    scratch_shapes=[pltpu.SMEM((2,), jnp.float32)],
)


def kernel(slot_bias_fid_index, label, sparse_bias, certain_bias,
           global_bias, W1, b1, W2, b2, W3, b3):
    bias_in, cert_in = _sc_gather(slot_bias_fid_index, sparse_bias,
                                  certain_bias)
    gb = (global_bias[0] + b3[0]).reshape(1, 1)
    pred, loss = _tower(bias_in, cert_in, label, gb, W1,
                        b1.reshape(1, D1), W2, b2.reshape(1, D2),
                        W3.reshape(1, D2))
    return pred, loss[0, 0]


# speedup vs baseline: 1.0826x; 1.0826x over previous
import functools

import jax
import jax.numpy as jnp
from jax import lax
from jax.experimental import pallas as pl
from jax.experimental.pallas import tpu as pltpu
from jax.experimental.pallas import tpu_sc as plsc

B = 16384
S = 100
D1 = 512
D2 = 256

NC = 2
NS = 16
NW = NC * NS
E = B * S
EW = E // NW
CE = EW // 4
NCHUNK = EW // CE

@functools.cache
def _get_sc_gather():
    mesh = plsc.VectorSubcoreMesh(core_axis_name="c", subcore_axis_name="s")

    @functools.partial(
        pl.kernel,
        mesh=mesh,
        out_type=(
            jax.ShapeDtypeStruct((E,), jnp.float32),
            jax.ShapeDtypeStruct((E,), jnp.float32),
        ),
        scratch_types=[
            pltpu.VMEM((CE,), jnp.int32),
            pltpu.VMEM((CE,), jnp.float32),
            pltpu.VMEM((CE,), jnp.float32),
            pltpu.SemaphoreType.DMA,
            pltpu.SemaphoreType.DMA,
        ],
    )
    def _sc_gather(idx_hbm, sparse_hbm, certain_hbm, out_s, out_c,
                   idx_v, sbuf, cbuf, sem_s, sem_c):
        wid = lax.axis_index("s") * NC + lax.axis_index("c")
        for k in range(NCHUNK):
            r0 = wid * EW + k * CE
            pltpu.sync_copy(idx_hbm.at[pl.ds(r0, CE)], idx_v)
            a = pltpu.async_copy(sparse_hbm.at[idx_v], sbuf, sem_s)
            b = pltpu.async_copy(certain_hbm.at[idx_v], cbuf, sem_c)
            a.wait()
            b.wait()
            pltpu.sync_copy(sbuf, out_s.at[pl.ds(r0, CE)])
            pltpu.sync_copy(cbuf, out_c.at[pl.ds(r0, CE)])

    return _sc_gather


BM = 1024


def _tower_body(x_ref, c_ref, lab_ref, gb_ref, w1_ref, b1_ref, w2_ref,
                b2_ref, w3_ref, pred_ref, loss_ref, acc_ref):
    i = pl.program_id(0)
    x = x_ref[...]
    h = jnp.dot(x, w1_ref[...], preferred_element_type=jnp.float32)
    h = jnp.maximum(h + b1_ref[...], 0.0)
    h = jnp.dot(h, w2_ref[...], preferred_element_type=jnp.float32)
    h = jnp.maximum(h + b2_ref[...], 0.0)
    nn = jnp.sum(h * w3_ref[...], axis=1)
    logits = jnp.sum(x, axis=1) + nn + gb_ref[0, 0]
    pred_ref[...] = jax.nn.sigmoid(logits)
    craw = jax.nn.sigmoid(jnp.sum(c_ref[...], axis=1)) + 0.5
    xent = (jnp.maximum(logits, 0.0) - logits * lab_ref[...]
            + jnp.log1p(jnp.exp(-jnp.abs(logits))))
    s0 = jnp.sum(xent * craw)
    s1 = jnp.sum(craw)

    @pl.when(i == 0)
    def _():
        acc_ref[0] = 0.0
        acc_ref[1] = 0.0

    acc_ref[0] += s0
    acc_ref[1] += s1

    @pl.when(i == pl.num_programs(0) - 1)
    def _():
        loss_ref[...] = (B * acc_ref[0] / acc_ref[1]) * jnp.ones(
            (1, 1), jnp.float32)


_tower = pl.pallas_call(
    _tower_body,
    grid=(B // BM,),
    in_specs=[
        pl.BlockSpec((BM, S), lambda i: (i, 0)),
        pl.BlockSpec((BM, S), lambda i: (i, 0)),
        pl.BlockSpec((BM,), lambda i: (i,)),
        pl.BlockSpec(memory_space=pltpu.SMEM),
        pl.BlockSpec((S, D1), lambda i: (0, 0)),
        pl.BlockSpec((1, D1), lambda i: (0, 0)),
        pl.BlockSpec((D1, D2), lambda i: (0, 0)),
        pl.BlockSpec((1, D2), lambda i: (0, 0)),
        pl.BlockSpec((1, D2), lambda i: (0, 0)),
    ],
    out_specs=[
        pl.BlockSpec((BM,), lambda i: (i,)),
        pl.BlockSpec((1, 1), lambda i: (0, 0)),
    ],
    out_shape=[
        jax.ShapeDtypeStruct((B,), jnp.float32),
        jax.ShapeDtypeStruct((1, 1), jnp.float32),
    ],
    scratch_shapes=[pltpu.SMEM((2,), jnp.float32)],
)


def kernel(slot_bias_fid_index, label, sparse_bias, certain_bias,
           global_bias, W1, b1, W2, b2, W3, b3):
    bias_f, cert_f = _get_sc_gather()(slot_bias_fid_index.reshape(-1),
                                      sparse_bias, certain_bias)
    bias_in = bias_f.reshape(B, S)
    cert_in = cert_f.reshape(B, S)
    gb = (global_bias[0] + b3[0]).reshape(1, 1)
    pred, loss = _tower(bias_in, cert_in, label, gb, W1,
                        b1.reshape(1, D1), W2, b2.reshape(1, D2),
                        W3.reshape(1, D2))
    return pred, loss[0, 0]


# speedup vs baseline: 1.1673x; 1.0782x over previous
import functools

import jax
import jax.numpy as jnp
from jax import lax
from jax.experimental import pallas as pl
from jax.experimental.pallas import tpu as pltpu
from jax.experimental.pallas import tpu_sc as plsc

B = 16384
S = 100
D1 = 512
D2 = 256

NSPLIT = 4
BCH = B // NSPLIT

NC = 2
NS = 16
NW = NC * NS
E = BCH * S
EW = E // NW
CE = EW


@functools.cache
def _get_sc_gather():
    mesh = plsc.VectorSubcoreMesh(core_axis_name="c", subcore_axis_name="s")

    @functools.partial(
        pl.kernel,
        mesh=mesh,
        out_type=(
            jax.ShapeDtypeStruct((E,), jnp.float32),
            jax.ShapeDtypeStruct((E,), jnp.float32),
        ),
        scratch_types=[
            pltpu.VMEM((CE,), jnp.int32),
            pltpu.VMEM((CE,), jnp.float32),
            pltpu.VMEM((CE,), jnp.float32),
            pltpu.SemaphoreType.DMA,
            pltpu.SemaphoreType.DMA,
        ],
    )
    def _sc_gather(idx_hbm, sparse_hbm, certain_hbm, out_s, out_c,
                   idx_v, sbuf, cbuf, sem_s, sem_c):
        wid = lax.axis_index("s") * NC + lax.axis_index("c")
        r0 = wid * EW
        pltpu.sync_copy(idx_hbm.at[pl.ds(r0, CE)], idx_v)
        a = pltpu.async_copy(sparse_hbm.at[idx_v], sbuf, sem_s)
        b = pltpu.async_copy(certain_hbm.at[idx_v], cbuf, sem_c)
        a.wait()
        b.wait()
        pltpu.sync_copy(sbuf, out_s.at[pl.ds(r0, CE)])
        pltpu.sync_copy(cbuf, out_c.at[pl.ds(r0, CE)])

    return _sc_gather


BM = 1024


def _tower_body(x_ref, c_ref, lab_ref, gb_ref, w1_ref, b1_ref, w2_ref,
                b2_ref, w3_ref, pred_ref, part_ref, acc_ref):
    i = pl.program_id(0)
    x = x_ref[...]
    h = jnp.dot(x, w1_ref[...], preferred_element_type=jnp.float32)
    h = jnp.maximum(h + b1_ref[...], 0.0)
    h = jnp.dot(h, w2_ref[...], preferred_element_type=jnp.float32)
    h = jnp.maximum(h + b2_ref[...], 0.0)
    nn = jnp.sum(h * w3_ref[...], axis=1)
    logits = jnp.sum(x, axis=1) + nn + gb_ref[0, 0]
    pred_ref[...] = jax.nn.sigmoid(logits)
    craw = jax.nn.sigmoid(jnp.sum(c_ref[...], axis=1)) + 0.5
    xent = (jnp.maximum(logits, 0.0) - logits * lab_ref[...]
            + jnp.log1p(jnp.exp(-jnp.abs(logits))))
    s0 = jnp.sum(xent * craw)
    s1 = jnp.sum(craw)

    @pl.when(i == 0)
    def _():
        acc_ref[0] = 0.0
        acc_ref[1] = 0.0

    acc_ref[0] += s0
    acc_ref[1] += s1

    @pl.when(i == pl.num_programs(0) - 1)
    def _():
        part_ref[0] = acc_ref[0]
        part_ref[1] = acc_ref[1]


_tower = pl.pallas_call(
    _tower_body,
    grid=(BCH // BM,),
    in_specs=[
        pl.BlockSpec((BM, S), lambda i: (i, 0)),
        pl.BlockSpec((BM, S), lambda i: (i, 0)),
        pl.BlockSpec((BM,), lambda i: (i,)),
        pl.BlockSpec(memory_space=pltpu.SMEM),
        pl.BlockSpec((S, D1), lambda i: (0, 0)),
        pl.BlockSpec((1, D1), lambda i: (0, 0)),
        pl.BlockSpec((D1, D2), lambda i: (0, 0)),
        pl.BlockSpec((1, D2), lambda i: (0, 0)),
        pl.BlockSpec((1, D2), lambda i: (0, 0)),
    ],
    out_specs=[
        pl.BlockSpec((BM,), lambda i: (i,)),
        pl.BlockSpec(memory_space=pltpu.SMEM),
    ],
    out_shape=[
        jax.ShapeDtypeStruct((BCH,), jnp.float32),
        jax.ShapeDtypeStruct((2,), jnp.float32),
    ],
    scratch_shapes=[pltpu.SMEM((2,), jnp.float32)],
)


def kernel(slot_bias_fid_index, label, sparse_bias, certain_bias,
           global_bias, W1, b1, W2, b2, W3, b3):
    idx_flat = slot_bias_fid_index.reshape(-1)
    gb = (global_bias[0] + b3[0]).reshape(1, 1)
    b1r = b1.reshape(1, D1)
    b2r = b2.reshape(1, D2)
    w3r = W3.reshape(1, D2)
    sc_gather = _get_sc_gather()
    preds = []
    parts = []
    for k in range(NSPLIT):
        bias_f, cert_f = sc_gather(
            lax.dynamic_slice(idx_flat, (k * E,), (E,)),
            sparse_bias, certain_bias)
        pred_k, part_k = _tower(bias_f.reshape(BCH, S),
                                cert_f.reshape(BCH, S),
                                lax.dynamic_slice(label, (k * BCH,), (BCH,)),
                                gb, W1, b1r, W2, b2r, w3r)
        preds.append(pred_k)
        parts.append(part_k)
    pred = jnp.concatenate(preds)
    ps = jnp.stack(parts)
    s0 = jnp.sum(ps[:, 0])
    s1 = jnp.sum(ps[:, 1])
    loss = B * s0 / s1
    return pred, loss
